# baseline (speedup 1.0000x reference)
import functools

import jax
import jax.numpy as jnp
from jax import lax
from jax.experimental import pallas as pl
from jax.experimental.pallas import tpu as pltpu
from jax.experimental.pallas import tpu_sc as plsc

N = 2048
M = N * (N - 1)
ROWS = N - 1
NC, NS = 2, 16
NW = NC * NS
CR = 4
CHUNKS = 16
CW = CR * N


def _compute_row(k, off, xv, yv, zv, xb, yb, zb, db, mb):
    ii = lax.broadcasted_iota(jnp.int32, (16,), 0)
    magic = jnp.full((16,), 0x5F3759DF, dtype=jnp.int32)
    half = jnp.full((16,), 0.5, dtype=jnp.float32)
    three_half = jnp.full((16,), 1.5, dtype=jnp.float32)
    cutoff2 = plsc.bitcast(jnp.full((16,), 0x3E23D70B, dtype=jnp.int32),
                           jnp.float32)
    one = jnp.full((16,), 1, dtype=jnp.int32)
    zero = jnp.full((16,), 0, dtype=jnp.int32)

    k1 = k + 1
    kv = jnp.full((16,), k, dtype=jnp.int32)
    k1v = kv + one
    xk = plsc.load_gather(xv, [kv])
    yk = plsc.load_gather(yv, [kv])
    zk = plsc.load_gather(zv, [kv])
    xk1 = plsc.load_gather(xv, [k1v])
    yk1 = plsc.load_gather(yv, [k1v])
    zk1 = plsc.load_gather(zv, [k1v])
    switch = N - 1 - k

    @plsc.parallel_loop(0, N // 16, step=1, unroll=8)
    def blk_body(b):
        c0 = b * 16
        cmp = (c0 + ii) >= switch
        xj = xv[pl.ds(k1 + c0, 16)]
        yj = yv[pl.ds(k1 + c0, 16)]
        zj = zv[pl.ds(k1 + c0, 16)]
        dx = jnp.where(cmp, xk1, xk) - xj
        dy = jnp.where(cmp, yk1, yk) - yj
        dz = jnp.where(cmp, zk1, zk) - zj
        d2 = dx * dx + dy * dy + dz * dz
        y = plsc.bitcast(magic - (plsc.bitcast(d2, jnp.int32) >> 1),
                         jnp.float32)
        hd2 = half * d2
        y = y * (three_half - hd2 * y * y)
        y = y * (three_half - hd2 * y * y)
        d = d2 * y
        m = (d2 <= cutoff2).astype(jnp.int32)
        xb[pl.ds(off + c0, 16)] = dx
        yb[pl.ds(off + c0, 16)] = dy
        zb[pl.ds(off + c0, 16)] = dz
        db[pl.ds(off + c0, 16)] = d
        mb[pl.ds(off + c0, 16)] = m


def _sc_body(x_hbm, y_hbm, z_hbm,
             r_hbm, d_hbm, m_hbm,
             xv, yv, zv,
             xb0, yb0, zb0, db0, mb0,
             xb1, yb1, zb1, db1, mb1,
             sem0, sem1):
    wid = lax.axis_index("s") * NC + lax.axis_index("c")
    row_base = wid * CR * CHUNKS

    pltpu.sync_copy(x_hbm, xv)
    pltpu.sync_copy(y_hbm, yv)
    pltpu.sync_copy(z_hbm, zv)

    slots = ((xb0, yb0, zb0, db0, mb0, sem0),
             (xb1, yb1, zb1, db1, mb1, sem1))

    def chunk(u, s):
        xb, yb, zb, db, mb, sem = slots[s]
        cc = 2 * u + s
        k0 = jnp.minimum(row_base + CR * cc, ROWS - CR)

        @pl.when(u > 0)
        def _wait():
            for buf in (xb, yb, zb, db, mb):
                pltpu.make_async_copy(buf, d_hbm.at[pl.ds(0, CW)], sem).wait()

        for r in range(CR):
            _compute_row(k0 + r, r * N, xv, yv, zv, xb, yb, zb, db, mb)

        base = k0 * N
        pltpu.async_copy(xb, r_hbm.at[pl.ds(base, CW)], sem)
        pltpu.async_copy(yb, r_hbm.at[pl.ds(M + base, CW)], sem)
        pltpu.async_copy(zb, r_hbm.at[pl.ds(2 * M + base, CW)], sem)
        pltpu.async_copy(db, d_hbm.at[pl.ds(base, CW)], sem)
        pltpu.async_copy(mb, m_hbm.at[pl.ds(base, CW)], sem)

    def u_body(u, _):
        chunk(u, 0)
        chunk(u, 1)
        return 0

    lax.fori_loop(0, CHUNKS // 2, u_body, 0)

    for s in (0, 1):
        xb, yb, zb, db, mb, sem = slots[s]
        for buf in (xb, yb, zb, db, mb):
            pltpu.make_async_copy(buf, d_hbm.at[pl.ds(0, CW)], sem).wait()


_sc_pairs = functools.partial(
    pl.kernel,
    mesh=plsc.VectorSubcoreMesh(core_axis_name="c", subcore_axis_name="s"),
    compiler_params=pltpu.CompilerParams(needs_layout_passes=False),
    out_type=[
        jax.ShapeDtypeStruct((3 * M,), jnp.float32),
        jax.ShapeDtypeStruct((M,), jnp.float32),
        jax.ShapeDtypeStruct((M,), jnp.int32),
    ],
    scratch_types=(
        [pltpu.VMEM((2 * N,), jnp.float32)] * 3
        + [pltpu.VMEM((CW,), jnp.float32)] * 4
        + [pltpu.VMEM((CW,), jnp.int32)]
        + [pltpu.VMEM((CW,), jnp.float32)] * 4
        + [pltpu.VMEM((CW,), jnp.int32)]
        + [pltpu.SemaphoreType.DMA] * 2
    ),
)(_sc_body)


def _pair_body(pair_ref):
    pid = pl.program_id(0)
    L = pair_ref.shape[1]
    p = lax.broadcasted_iota(jnp.int32, (1, L), 1) + pid * L
    k = p >> 11
    f = p + k + 1
    i = f >> 11
    j = f & (N - 1)
    pair_ref[...] = jnp.concatenate([i, j], axis=0)


_PAIR_L = 131072


def _pair_indices():
    return pl.pallas_call(
        _pair_body,
        grid=(pl.cdiv(M, _PAIR_L),),
        out_specs=pl.BlockSpec((2, _PAIR_L), lambda g: (0, g)),
        out_shape=jax.ShapeDtypeStruct((2, M), jnp.int32),
    )()


def kernel(positions, atomic_subsystem_indices):
    comp = jnp.concatenate([positions.T, positions.T], axis=1)
    pair = _pair_indices()
    rxyz, d, mask = _sc_pairs(comp[0], comp[1], comp[2])
    r = jnp.stack([rxyz[:M], rxyz[M:2 * M], rxyz[2 * M:]], axis=1)
    return (
        pair,
        d.reshape(M, 1),
        r,
        mask.astype(jnp.bool_).reshape(M, 1),
    )


# speedup vs baseline: 1.2147x; 1.2147x over previous
import functools

import jax
import jax.numpy as jnp
from jax import lax
from jax.experimental import pallas as pl
from jax.experimental.pallas import tpu as pltpu
from jax.experimental.pallas import tpu_sc as plsc

N = 2048
M = N * (N - 1)
ROWS = N - 1
NC, NS = 2, 16
NW = NC * NS
CR = 4
CHUNKS = 16
CW = CR * N


def _compute_row(k, off, xv, yv, zv, xb, yb, zb, db, mb):
    ii = lax.broadcasted_iota(jnp.int32, (16,), 0)
    magic = jnp.full((16,), 0x5F3759DF, dtype=jnp.int32)
    half = jnp.full((16,), 0.5, dtype=jnp.float32)
    three_half = jnp.full((16,), 1.5, dtype=jnp.float32)
    cutoff2 = plsc.bitcast(jnp.full((16,), 0x3E23D70B, dtype=jnp.int32),
                           jnp.float32)
    one = jnp.full((16,), 1, dtype=jnp.int32)

    k1 = k + 1
    kv = jnp.full((16,), k, dtype=jnp.int32)
    k1v = kv + one
    xk = plsc.load_gather(xv, [kv])
    yk = plsc.load_gather(yv, [kv])
    zk = plsc.load_gather(zv, [kv])
    xk1 = plsc.load_gather(xv, [k1v])
    yk1 = plsc.load_gather(yv, [k1v])
    zk1 = plsc.load_gather(zv, [k1v])
    switch = N - 1 - k

    @plsc.parallel_loop(0, N // 16, step=1, unroll=8)
    def blk_body(b):
        c0 = b * 16
        cmp = (c0 + ii) >= switch
        xj = xv[pl.ds(k1 + c0, 16)]
        yj = yv[pl.ds(k1 + c0, 16)]
        zj = zv[pl.ds(k1 + c0, 16)]
        dx = jnp.where(cmp, xk1, xk) - xj
        dy = jnp.where(cmp, yk1, yk) - yj
        dz = jnp.where(cmp, zk1, zk) - zj
        d2 = dx * dx + dy * dy + dz * dz
        y = plsc.bitcast(magic - (plsc.bitcast(d2, jnp.int32) >> 1),
                         jnp.float32)
        hd2 = half * d2
        y = y * (three_half - hd2 * y * y)
        y = y * (three_half - hd2 * y * y)
        d = d2 * y
        m = (d2 <= cutoff2).astype(jnp.int32)
        xb[pl.ds(off + c0, 16)] = dx
        yb[pl.ds(off + c0, 16)] = dy
        zb[pl.ds(off + c0, 16)] = dz
        db[pl.ds(off + c0, 16)] = d
        mb[pl.ds(off + c0, 16)] = m


def _sc_body(x_hbm, y_hbm, z_hbm,
             xo_hbm, yo_hbm, zo_hbm, d_hbm, m_hbm,
             xv, yv, zv,
             xb0, yb0, zb0, db0, mb0,
             xb1, yb1, zb1, db1, mb1,
             sem0, sem1):
    wid = lax.axis_index("s") * NC + lax.axis_index("c")
    row_base = wid * CR * CHUNKS

    pltpu.sync_copy(x_hbm, xv)
    pltpu.sync_copy(y_hbm, yv)
    pltpu.sync_copy(z_hbm, zv)

    slots = ((xb0, yb0, zb0, db0, mb0, sem0),
             (xb1, yb1, zb1, db1, mb1, sem1))

    def chunk(u, s):
        xb, yb, zb, db, mb, sem = slots[s]
        cc = 2 * u + s
        k0 = jnp.minimum(row_base + CR * cc, ROWS - CR)

        @pl.when(u > 0)
        def _wait():
            for buf, hbm in ((xb, xo_hbm), (yb, yo_hbm), (zb, zo_hbm),
                             (db, d_hbm), (mb, m_hbm)):
                pltpu.make_async_copy(buf, hbm.at[pl.ds(0, CW)], sem).wait()

        for r in range(CR):
            _compute_row(k0 + r, r * N, xv, yv, zv, xb, yb, zb, db, mb)

        base = k0 * N
        pltpu.async_copy(xb, xo_hbm.at[pl.ds(base, CW)], sem)
        pltpu.async_copy(yb, yo_hbm.at[pl.ds(base, CW)], sem)
        pltpu.async_copy(zb, zo_hbm.at[pl.ds(base, CW)], sem)
        pltpu.async_copy(db, d_hbm.at[pl.ds(base, CW)], sem)
        pltpu.async_copy(mb, m_hbm.at[pl.ds(base, CW)], sem)

    def u_body(u, _):
        chunk(u, 0)
        chunk(u, 1)
        return 0

    lax.fori_loop(0, CHUNKS // 2, u_body, 0)

    for s in (0, 1):
        xb, yb, zb, db, mb, sem = slots[s]
        for buf, hbm in ((xb, xo_hbm), (yb, yo_hbm), (zb, zo_hbm),
                         (db, d_hbm), (mb, m_hbm)):
            pltpu.make_async_copy(buf, hbm.at[pl.ds(0, CW)], sem).wait()


_sc_pairs = functools.partial(
    pl.kernel,
    mesh=plsc.VectorSubcoreMesh(core_axis_name="c", subcore_axis_name="s"),
    compiler_params=pltpu.CompilerParams(needs_layout_passes=False),
    out_type=[
        jax.ShapeDtypeStruct((M,), jnp.float32),
        jax.ShapeDtypeStruct((M,), jnp.float32),
        jax.ShapeDtypeStruct((M,), jnp.float32),
        jax.ShapeDtypeStruct((M,), jnp.float32),
        jax.ShapeDtypeStruct((M,), jnp.int32),
    ],
    scratch_types=(
        [pltpu.VMEM((2 * N,), jnp.float32)] * 3
        + [pltpu.VMEM((CW,), jnp.float32)] * 4
        + [pltpu.VMEM((CW,), jnp.int32)]
        + [pltpu.VMEM((CW,), jnp.float32)] * 4
        + [pltpu.VMEM((CW,), jnp.int32)]
        + [pltpu.SemaphoreType.DMA] * 2
    ),
)(_sc_body)


def _pair_body(pair_ref):
    pid = pl.program_id(0)
    L = pair_ref.shape[1]
    p = lax.broadcasted_iota(jnp.int32, (1, L), 1) + pid * L
    k = p >> 11
    f = p + k + 1
    i = f >> 11
    j = f & (N - 1)
    pair_ref[...] = jnp.concatenate([i, j], axis=0)


_PAIR_L = 131072


def _pair_indices():
    return pl.pallas_call(
        _pair_body,
        grid=(pl.cdiv(M, _PAIR_L),),
        out_specs=pl.BlockSpec((2, _PAIR_L), lambda g: (0, g)),
        out_shape=jax.ShapeDtypeStruct((2, M), jnp.int32),
    )()


def kernel(positions, atomic_subsystem_indices):
    comp = jnp.concatenate([positions.T, positions.T], axis=1)
    pair = _pair_indices()
    rx, ry, rz, d, mask = _sc_pairs(comp[0], comp[1], comp[2])
    r = jnp.stack([rx, ry, rz], axis=1)
    return (
        pair,
        d.reshape(M, 1),
        r,
        mask.astype(jnp.bool_).reshape(M, 1),
    )


# speedup vs baseline: 1.2225x; 1.0064x over previous
import functools

import jax
import jax.numpy as jnp
from jax import lax
from jax.experimental import pallas as pl
from jax.experimental.pallas import tpu as pltpu
from jax.experimental.pallas import tpu_sc as plsc

N = 2048
M = N * (N - 1)
ROWS = N - 1
NC, NS = 2, 16
NW = NC * NS
CR = 2
CHUNKS = 32
CW = CR * N


def _compute_row(k, off, xv, yv, zv, xb, yb, zb, db, mb):
    ii = lax.broadcasted_iota(jnp.int32, (16,), 0)
    magic = jnp.full((16,), 0x5F3759DF, dtype=jnp.int32)
    half = jnp.full((16,), 0.5, dtype=jnp.float32)
    three_half = jnp.full((16,), 1.5, dtype=jnp.float32)
    cutoff2 = plsc.bitcast(jnp.full((16,), 0x3E23D70B, dtype=jnp.int32),
                           jnp.float32)
    one = jnp.full((16,), 1, dtype=jnp.int32)

    k1 = k + 1
    kv = jnp.full((16,), k, dtype=jnp.int32)
    k1v = kv + one
    xk = plsc.load_gather(xv, [kv])
    yk = plsc.load_gather(yv, [kv])
    zk = plsc.load_gather(zv, [kv])
    xk1 = plsc.load_gather(xv, [k1v])
    yk1 = plsc.load_gather(yv, [k1v])
    zk1 = plsc.load_gather(zv, [k1v])
    switch = N - 1 - k

    @plsc.parallel_loop(0, N // 16, step=1, unroll=8)
    def blk_body(b):
        c0 = b * 16
        cmp = (c0 + ii) >= switch
        xj = xv[pl.ds(k1 + c0, 16)]
        yj = yv[pl.ds(k1 + c0, 16)]
        zj = zv[pl.ds(k1 + c0, 16)]
        dx = jnp.where(cmp, xk1, xk) - xj
        dy = jnp.where(cmp, yk1, yk) - yj
        dz = jnp.where(cmp, zk1, zk) - zj
        d2 = dx * dx + dy * dy + dz * dz
        y = plsc.bitcast(magic - (plsc.bitcast(d2, jnp.int32) >> 1),
                         jnp.float32)
        hd2 = half * d2
        y = y * (three_half - hd2 * y * y)
        y = y * (three_half - hd2 * y * y)
        d = d2 * y
        m = (d2 <= cutoff2).astype(jnp.int32)
        xb[pl.ds(off + c0, 16)] = dx
        yb[pl.ds(off + c0, 16)] = dy
        zb[pl.ds(off + c0, 16)] = dz
        db[pl.ds(off + c0, 16)] = d
        mb[pl.ds(off + c0, 16)] = m


def _sc_body(x_hbm, y_hbm, z_hbm,
             xo_hbm, yo_hbm, zo_hbm, d_hbm, m_hbm,
             xv, yv, zv,
             xb0, yb0, zb0, db0, mb0,
             xb1, yb1, zb1, db1, mb1,
             sem0, sem1):
    wid = lax.axis_index("s") * NC + lax.axis_index("c")
    row_base = wid * CR * CHUNKS

    pltpu.sync_copy(x_hbm, xv)
    pltpu.sync_copy(y_hbm, yv)
    pltpu.sync_copy(z_hbm, zv)

    slots = ((xb0, yb0, zb0, db0, mb0, sem0),
             (xb1, yb1, zb1, db1, mb1, sem1))

    def chunk(u, s):
        xb, yb, zb, db, mb, sem = slots[s]
        cc = 2 * u + s
        k0 = jnp.minimum(row_base + CR * cc, ROWS - CR)

        @pl.when(u > 0)
        def _wait():
            for buf, hbm in ((xb, xo_hbm), (yb, yo_hbm), (zb, zo_hbm),
                             (db, d_hbm), (mb, m_hbm)):
                pltpu.make_async_copy(buf, hbm.at[pl.ds(0, CW)], sem).wait()

        for r in range(CR):
            _compute_row(k0 + r, r * N, xv, yv, zv, xb, yb, zb, db, mb)

        base = k0 * N
        pltpu.async_copy(xb, xo_hbm.at[pl.ds(base, CW)], sem)
        pltpu.async_copy(yb, yo_hbm.at[pl.ds(base, CW)], sem)
        pltpu.async_copy(zb, zo_hbm.at[pl.ds(base, CW)], sem)
        pltpu.async_copy(db, d_hbm.at[pl.ds(base, CW)], sem)
        pltpu.async_copy(mb, m_hbm.at[pl.ds(base, CW)], sem)

    def u_body(u, _):
        chunk(u, 0)
        chunk(u, 1)
        return 0

    lax.fori_loop(0, CHUNKS // 2, u_body, 0)

    for s in (0, 1):
        xb, yb, zb, db, mb, sem = slots[s]
        for buf, hbm in ((xb, xo_hbm), (yb, yo_hbm), (zb, zo_hbm),
                         (db, d_hbm), (mb, m_hbm)):
            pltpu.make_async_copy(buf, hbm.at[pl.ds(0, CW)], sem).wait()


_sc_pairs = functools.partial(
    pl.kernel,
    mesh=plsc.VectorSubcoreMesh(core_axis_name="c", subcore_axis_name="s"),
    compiler_params=pltpu.CompilerParams(needs_layout_passes=False),
    out_type=[
        jax.ShapeDtypeStruct((M,), jnp.float32),
        jax.ShapeDtypeStruct((M,), jnp.float32),
        jax.ShapeDtypeStruct((M,), jnp.float32),
        jax.ShapeDtypeStruct((M,), jnp.float32),
        jax.ShapeDtypeStruct((M,), jnp.int32),
    ],
    scratch_types=(
        [pltpu.VMEM((2 * N,), jnp.float32)] * 3
        + [pltpu.VMEM((CW,), jnp.float32)] * 4
        + [pltpu.VMEM((CW,), jnp.int32)]
        + [pltpu.VMEM((CW,), jnp.float32)] * 4
        + [pltpu.VMEM((CW,), jnp.int32)]
        + [pltpu.SemaphoreType.DMA] * 2
    ),
)(_sc_body)


def _pair_body(pair_ref):
    pid = pl.program_id(0)
    L = pair_ref.shape[1]
    p = lax.broadcasted_iota(jnp.int32, (1, L), 1) + pid * L
    k = p >> 11
    f = p + k + 1
    i = f >> 11
    j = f & (N - 1)
    pair_ref[...] = jnp.concatenate([i, j], axis=0)


_PAIR_L = 131072


def _pair_indices():
    return pl.pallas_call(
        _pair_body,
        grid=(pl.cdiv(M, _PAIR_L),),
        out_specs=pl.BlockSpec((2, _PAIR_L), lambda g: (0, g)),
        out_shape=jax.ShapeDtypeStruct((2, M), jnp.int32),
    )()


def kernel(positions, atomic_subsystem_indices):
    comp = jnp.concatenate([positions.T, positions.T], axis=1)
    pair = _pair_indices()
    rx, ry, rz, d, mask = _sc_pairs(comp[0], comp[1], comp[2])
    r = jnp.stack([rx, ry, rz], axis=1)
    return (
        pair,
        d.reshape(M, 1),
        r,
        mask.astype(jnp.bool_).reshape(M, 1),
    )
